# baseline (speedup 1.0000x reference)
import functools

import jax
import jax.numpy as jnp
from jax import lax
from jax.experimental import pallas as pl
from jax.experimental.pallas import tpu as pltpu
from jax.experimental.pallas import tpu_sc as plsc

B, S1, S2, K = 1024, 1, 32, 17
D = 128
W64 = D // 2
N = B * S1 * S2
NNEI = K - 1

NC, NS = 2, 16
NW = NC * NS
PPW = N // NW
P = 8
NCH = PPW // P
SCHUNK = 128
NSCH = PPW // SCHUNK
LANES = 16


def _sc_body(emb_h, idxn_h, idxs_h, self_out, nei_out,
             idxn_v, idxs_v, rows, nbuf, srows, gsem, osem, ssem, sosem):
    cid = lax.axis_index("c")
    sid = lax.axis_index("s")
    w = sid * NC + cid
    base = w * PPW

    pltpu.sync_copy(idxn_h.at[pl.ds(w * NCH, NCH)], idxn_v)
    pltpu.sync_copy(idxs_h.at[pl.ds(w * NSCH, NSCH)], idxs_v)

    def fire_nei(c, slot):
        pltpu.async_copy(emb_h.at[idxn_v.at[c]], rows.at[slot], gsem.at[slot])

    def wait_nei(c, slot):
        pltpu.make_async_copy(emb_h.at[idxn_v.at[c]], rows.at[slot],
                              gsem.at[slot]).wait()

    def reduce_chunk(slot, c):
        hi_mask = jnp.int32(-65536)

        def pos(p, carry):
            for d in range(W64 // LANES):
                w0 = rows[slot, p * NNEI, pl.ds(d * LANES, LANES)]
                acc_e = lax.bitcast_convert_type(w0 << 16, jnp.float32)
                acc_o = lax.bitcast_convert_type(w0 & hi_mask, jnp.float32)
                for r in range(1, NNEI):
                    wr = rows[slot, p * NNEI + r, pl.ds(d * LANES, LANES)]
                    acc_e = acc_e + lax.bitcast_convert_type(wr << 16,
                                                             jnp.float32)
                    acc_o = acc_o + lax.bitcast_convert_type(wr & hi_mask,
                                                             jnp.float32)
                nbuf[slot, p, pl.ds(2 * d * LANES, LANES)] = acc_e
                nbuf[slot, p, pl.ds((2 * d + 1) * LANES, LANES)] = acc_o
            return carry
        lax.fori_loop(0, P, pos, 0)

    fire_nei(0, 0)
    fire_nei(1, 1)

    def process(i, c, slot):
        wait_nei(c, slot)

        @pl.when(i > 0)
        def _():
            pltpu.make_async_copy(nbuf.at[slot],
                                  nei_out.at[pl.ds(base, P)],
                                  osem.at[slot]).wait()

        reduce_chunk(slot, c)
        pltpu.async_copy(nbuf.at[slot],
                         nei_out.at[pl.ds(base + c * P, P)],
                         osem.at[slot])

        @pl.when(i < NCH // 2 - 1)
        def _():
            fire_nei(c + 2, slot)

    def pair_body(i, carry):
        process(i, 2 * i, 0)
        process(i, 2 * i + 1, 1)
        return carry

    lax.fori_loop(0, NCH // 2, pair_body, 0)

    pltpu.make_async_copy(nbuf.at[0], nei_out.at[pl.ds(base, P)],
                          osem.at[0]).wait()
    pltpu.make_async_copy(nbuf.at[1], nei_out.at[pl.ds(base, P)],
                          osem.at[1]).wait()

    def fire_self(j, slot):
        pltpu.async_copy(emb_h.at[idxs_v.at[j]], srows.at[slot], ssem.at[slot])

    fire_self(0, 0)
    fire_self(1, 1)
    for j in range(NSCH):
        slot = j & 1
        pltpu.make_async_copy(emb_h.at[idxs_v.at[j]], srows.at[slot],
                              ssem.at[slot]).wait()
        dst = self_out.at[pl.ds(base + j * SCHUNK, SCHUNK)]
        pltpu.async_copy(srows.at[slot], dst, sosem.at[slot])
        pltpu.make_async_copy(srows.at[slot], dst, sosem.at[slot]).wait()
        if j + 2 < NSCH:
            fire_self(j + 2, slot)


_sc_gather_reduce = functools.partial(
    pl.kernel,
    out_type=(jax.ShapeDtypeStruct((N, W64), jnp.int32),
              jax.ShapeDtypeStruct((N, D), jnp.float32)),
    mesh=plsc.VectorSubcoreMesh(core_axis_name="c", subcore_axis_name="s"),
    compiler_params=pltpu.CompilerParams(use_tc_tiling_on_sc=False),
    scratch_types=[
        pltpu.VMEM((NCH, 128), jnp.int32),
        pltpu.VMEM((NSCH, 128), jnp.int32),
        pltpu.VMEM((2, P * NNEI, W64), jnp.int32),
        pltpu.VMEM((2, P, D), jnp.float32),
        pltpu.VMEM((2, SCHUNK, W64), jnp.int32),
        pltpu.SemaphoreType.DMA((2,)),
        pltpu.SemaphoreType.DMA((2,)),
        pltpu.SemaphoreType.DMA((2,)),
        pltpu.SemaphoreType.DMA((2,)),
    ],
)(_sc_body)

import numpy as _np

_PERM = _np.concatenate(
    [_np.concatenate([32 * g + _np.arange(0, 32, 2),
                      32 * g + _np.arange(1, 32, 2)])
     for g in range(D // 32)])


def _mm_body(self_ref, nei_ref, w0a, w0s, w1a, w1s, out_ref):
    aggr = nei_ref[...].astype(jnp.float32) * (1.0 / NNEI)
    selfx = self_ref[...].astype(jnp.float32)
    h = jnp.maximum(
        jnp.dot(selfx, w0s[...], preferred_element_type=jnp.float32)
        + jnp.dot(aggr, w0a[...], preferred_element_type=jnp.float32), 0.0)
    out_ref[...] = jnp.maximum(
        jnp.dot(h, w1s[...], preferred_element_type=jnp.float32)
        + jnp.dot(aggr, w1a[...], preferred_element_type=jnp.float32), 0.0)


_MM_R = 4096
_mm = pl.pallas_call(
    _mm_body,
    out_shape=jax.ShapeDtypeStruct((N, D), jnp.float32),
    grid=(N // _MM_R,),
    in_specs=[
        pl.BlockSpec((_MM_R, D), lambda i: (i, 0)),
        pl.BlockSpec((_MM_R, D), lambda i: (i, 0)),
        pl.BlockSpec((D, D), lambda i: (0, 0)),
        pl.BlockSpec((D, D), lambda i: (0, 0)),
        pl.BlockSpec((D, D), lambda i: (0, 0)),
        pl.BlockSpec((D, D), lambda i: (0, 0)),
    ],
    out_specs=pl.BlockSpec((_MM_R, D), lambda i: (i, 0)),
)


def kernel(adj_org, Emb, W0_agg, W0_self, W1_agg, W1_self):
    vocab = Emb.shape[0]
    emb_i32 = lax.bitcast_convert_type(
        Emb.astype(jnp.bfloat16).reshape(vocab, W64, 2), jnp.int32)
    adj = adj_org.reshape(N, K).astype(jnp.int32)
    idx_self = adj[:, 0].reshape(N // 128, 128)
    idx_nei = adj[:, 1:].reshape(N * NNEI // 128, 128)
    self_i32, nei_sum = _sc_gather_reduce(emb_i32, idx_nei, idx_self)
    self_bf = lax.bitcast_convert_type(self_i32, jnp.bfloat16).reshape(N, D)
    out = _mm(self_bf, nei_sum, W0_agg[_PERM], W0_self, W1_agg[_PERM],
              W1_self)
    return out.reshape(B, S1, S2, D)


# speedup vs baseline: 2.4652x; 2.4652x over previous
import functools

import jax
import jax.numpy as jnp
from jax import lax
from jax.experimental import pallas as pl
from jax.experimental.pallas import tpu as pltpu
from jax.experimental.pallas import tpu_sc as plsc

B, S1, S2, K = 1024, 1, 32, 17
D = 128
W64 = D // 2
N = B * S1 * S2
NNEI = K - 1

NC, NS = 2, 16
NW = NC * NS
PPW = N // NW
P = 8
NCH = PPW // P
SCHUNK = 128
NSCH = PPW // SCHUNK
LANES = 16


def _sc_body(emb_h, idxn_h, idxs_h, self_out, nei_out,
             idxn_v, idxs_v, rows, nbuf, srows, sbuf,
             gsem, osem, ssem, sosem):
    _HI = jnp.int32(-65536)
    cid = lax.axis_index("c")
    sid = lax.axis_index("s")
    w = sid * NC + cid
    base = w * PPW
    tbase = base // P

    pltpu.sync_copy(idxn_h.at[pl.ds(base * NNEI, PPW * NNEI)], idxn_v)
    pltpu.sync_copy(idxs_h.at[pl.ds(base, PPW)], idxs_v)

    def fire_nei(c, slot):
        pltpu.async_copy(emb_h.at[idxn_v.at[pl.ds(c * 128, 128)]],
                         rows.at[slot], gsem.at[slot])

    def wait_nei(c, slot):
        pltpu.make_async_copy(emb_h.at[idxn_v.at[pl.ds(c * 128, 128)]],
                              rows.at[slot], gsem.at[slot]).wait()

    def reduce_chunk(slot, c):
        def pos(p, carry):
            for d in range(W64 // LANES):
                w0 = rows[slot, p * NNEI, pl.ds(d * LANES, LANES)]
                acc_lo = lax.bitcast_convert_type(w0 << 16, jnp.float32)
                acc_hi = lax.bitcast_convert_type(w0 & _HI, jnp.float32)
                for r in range(1, NNEI):
                    wr = rows[slot, p * NNEI + r, pl.ds(d * LANES, LANES)]
                    acc_lo = acc_lo + lax.bitcast_convert_type(
                        wr << 16, jnp.float32)
                    acc_hi = acc_hi + lax.bitcast_convert_type(
                        wr & _HI, jnp.float32)
                nbuf[slot, p, pl.ds(d * LANES, LANES)] = acc_lo
                nbuf[slot, p, pl.ds(W64 + d * LANES, LANES)] = acc_hi
            return carry
        lax.fori_loop(0, P, pos, 0)

    fire_nei(0, 0)
    fire_nei(1, 1)

    def process(i, c, slot):
        wait_nei(c, slot)

        @pl.when(i > 0)
        def _():
            pltpu.make_async_copy(nbuf.at[slot], nei_out.at[tbase],
                                  osem.at[slot]).wait()

        reduce_chunk(slot, c)
        pltpu.async_copy(nbuf.at[slot], nei_out.at[tbase + c],
                         osem.at[slot])

        @pl.when(i < NCH // 2 - 1)
        def _():
            fire_nei(c + 2, slot)

    def pair_body(i, carry):
        process(i, 2 * i, 0)
        process(i, 2 * i + 1, 1)
        return carry

    lax.fori_loop(0, NCH // 2, pair_body, 0)

    pltpu.make_async_copy(nbuf.at[0], nei_out.at[tbase], osem.at[0]).wait()
    pltpu.make_async_copy(nbuf.at[1], nei_out.at[tbase], osem.at[1]).wait()

    def fire_self(j, slot):
        pltpu.async_copy(emb_h.at[idxs_v.at[pl.ds(j * SCHUNK, SCHUNK)]],
                         srows.at[slot], ssem.at[slot])

    def decode_self(slot):
        def grp(g, carry):
            for r8 in range(8):
                for d in range(W64 // LANES):
                    wv = srows[slot, g * 8 + r8, pl.ds(d * LANES, LANES)]
                    sbuf[slot, g, r8, pl.ds(d * LANES, LANES)] = (
                        lax.bitcast_convert_type(wv << 16, jnp.float32))
                    sbuf[slot, g, r8, pl.ds(W64 + d * LANES, LANES)] = (
                        lax.bitcast_convert_type(wv & _HI, jnp.float32))
            return carry
        lax.fori_loop(0, SCHUNK // 8, grp, 0)

    fire_self(0, 0)
    fire_self(1, 1)
    for j in range(NSCH):
        slot = j & 1
        pltpu.make_async_copy(emb_h.at[idxs_v.at[pl.ds(j * SCHUNK, SCHUNK)]],
                              srows.at[slot], ssem.at[slot]).wait()
        if j >= 2:
            pltpu.make_async_copy(
                sbuf.at[slot],
                self_out.at[pl.ds(tbase, SCHUNK // 8)],
                sosem.at[slot]).wait()
        decode_self(slot)
        pltpu.async_copy(
            sbuf.at[slot],
            self_out.at[pl.ds(tbase + j * (SCHUNK // 8), SCHUNK // 8)],
            sosem.at[slot])
        if j + 2 < NSCH:
            fire_self(j + 2, slot)
    for slot in range(2):
        pltpu.make_async_copy(sbuf.at[slot],
                              self_out.at[pl.ds(tbase, SCHUNK // 8)],
                              sosem.at[slot]).wait()


_sc_gather_reduce = functools.partial(
    pl.kernel,
    out_type=(jax.ShapeDtypeStruct((N // 8, 8, D), jnp.float32),
              jax.ShapeDtypeStruct((N // 8, 8, D), jnp.float32)),
    mesh=plsc.VectorSubcoreMesh(core_axis_name="c", subcore_axis_name="s"),
    compiler_params=pltpu.CompilerParams(use_tc_tiling_on_sc=False),
    scratch_types=[
        pltpu.VMEM((PPW * NNEI,), jnp.int32),
        pltpu.VMEM((PPW,), jnp.int32),
        pltpu.VMEM((2, P * NNEI, W64), jnp.int32),
        pltpu.VMEM((2, P, D), jnp.float32),
        pltpu.VMEM((2, SCHUNK, W64), jnp.int32),
        pltpu.VMEM((2, SCHUNK // 8, 8, D), jnp.float32),
        pltpu.SemaphoreType.DMA((2,)),
        pltpu.SemaphoreType.DMA((2,)),
        pltpu.SemaphoreType.DMA((2,)),
        pltpu.SemaphoreType.DMA((2,)),
    ],
)(_sc_body)


def _mm_body(self_ref, nei_ref, w0a, w0s, w1a, w1s, out_ref):
    blk = out_ref.shape[0]
    aggr = nei_ref[...].reshape(blk, D) * (1.0 / NNEI)
    selfx = self_ref[...].reshape(blk, D)
    h = jnp.maximum(
        jnp.dot(selfx, w0s[...], preferred_element_type=jnp.float32)
        + jnp.dot(aggr, w0a[...], preferred_element_type=jnp.float32), 0.0)
    out_ref[...] = jnp.maximum(
        jnp.dot(h, w1s[...], preferred_element_type=jnp.float32)
        + jnp.dot(aggr, w1a[...], preferred_element_type=jnp.float32), 0.0)


_MM_R = 4096
_mm = pl.pallas_call(
    _mm_body,
    out_shape=jax.ShapeDtypeStruct((N, D), jnp.float32),
    grid=(N // _MM_R,),
    in_specs=[
        pl.BlockSpec((_MM_R // 8, 8, D), lambda i: (i, 0, 0)),
        pl.BlockSpec((_MM_R // 8, 8, D), lambda i: (i, 0, 0)),
        pl.BlockSpec((D, D), lambda i: (0, 0)),
        pl.BlockSpec((D, D), lambda i: (0, 0)),
        pl.BlockSpec((D, D), lambda i: (0, 0)),
        pl.BlockSpec((D, D), lambda i: (0, 0)),
    ],
    out_specs=pl.BlockSpec((_MM_R, D), lambda i: (i, 0)),
)


def kernel(adj_org, Emb, W0_agg, W0_self, W1_agg, W1_self):
    lo = lax.bitcast_convert_type(
        Emb[:, :W64].astype(jnp.bfloat16), jnp.uint16).astype(jnp.uint32)
    hi = lax.bitcast_convert_type(
        Emb[:, W64:].astype(jnp.bfloat16), jnp.uint16).astype(jnp.uint32)
    emb_i32 = lax.bitcast_convert_type(lo | (hi << 16), jnp.int32)

    adj = adj_org.reshape(N, K).astype(jnp.int32)
    idx_self = adj[:, 0]
    idx_nei = adj[:, 1:].reshape(N * NNEI)
    self_rows, nei_sum = _sc_gather_reduce(emb_i32, idx_nei, idx_self)
    out = _mm(self_rows, nei_sum, W0_agg, W0_self, W1_agg, W1_self)
    return out.reshape(B, S1, S2, D)


# speedup vs baseline: 2.5219x; 1.0230x over previous
import functools

import jax
import jax.numpy as jnp
from jax import lax
from jax.experimental import pallas as pl
from jax.experimental.pallas import tpu as pltpu
from jax.experimental.pallas import tpu_sc as plsc

B, S1, S2, K = 1024, 1, 32, 17
D = 128
W64 = D // 2
N = B * S1 * S2
NNEI = K - 1

NC, NS = 2, 16
NW = NC * NS
PPW = N // NW
P = 8
NCH = PPW // P
SCHUNK = 128
NSCH = PPW // SCHUNK
LANES = 16


def _sc_self_body(emb_h, idxs_h, self_out, idxs_v, srows, ssem, sosem):
    cid = lax.axis_index("c")
    sid = lax.axis_index("s")
    w = sid * NC + cid
    base = w * PPW

    pltpu.sync_copy(idxs_h.at[pl.ds(w * NSCH, NSCH)], idxs_v)

    def fire_self(j, slot):
        pltpu.async_copy(emb_h.at[idxs_v.at[j]], srows.at[slot], ssem.at[slot])

    fire_self(0, 0)
    fire_self(1, 1)
    for j in range(NSCH):
        slot = j & 1
        pltpu.make_async_copy(emb_h.at[idxs_v.at[j]], srows.at[slot],
                              ssem.at[slot]).wait()
        dst = self_out.at[pl.ds(base + j * SCHUNK, SCHUNK)]
        pltpu.async_copy(srows.at[slot], dst, sosem.at[slot])
        pltpu.make_async_copy(srows.at[slot], dst, sosem.at[slot]).wait()
        if j + 2 < NSCH:
            fire_self(j + 2, slot)


_sc_self = functools.partial(
    pl.kernel,
    out_type=jax.ShapeDtypeStruct((N, D), jnp.float32),
    mesh=plsc.VectorSubcoreMesh(core_axis_name="c", subcore_axis_name="s"),
    scratch_types=[
        pltpu.VMEM((NSCH, 128), jnp.int32),
        pltpu.VMEM((2, SCHUNK, D), jnp.float32),
        pltpu.SemaphoreType.DMA((2,)),
        pltpu.SemaphoreType.DMA((2,)),
    ],
)(_sc_self_body)


def _sc_nei_body(emb_h, idxn_h, nei_out, idxn_v, rows, nbuf, gsem, osem):
    _HI = jnp.int32(-65536)
    cid = lax.axis_index("c")
    sid = lax.axis_index("s")
    w = sid * NC + cid
    base = w * PPW
    tbase = base // P

    pltpu.sync_copy(idxn_h.at[pl.ds(base * NNEI, PPW * NNEI)], idxn_v)

    def fire_nei(c, slot):
        pltpu.async_copy(emb_h.at[idxn_v.at[pl.ds(c * 128, 128)]],
                         rows.at[slot], gsem.at[slot])

    def wait_nei(c, slot):
        pltpu.make_async_copy(emb_h.at[idxn_v.at[pl.ds(c * 128, 128)]],
                              rows.at[slot], gsem.at[slot]).wait()

    def reduce_chunk(slot, c):
        def pos(p, carry):
            for d in range(W64 // LANES):
                w0 = rows[slot, p * NNEI, pl.ds(d * LANES, LANES)]
                acc_lo = lax.bitcast_convert_type(w0 << 16, jnp.float32)
                acc_hi = lax.bitcast_convert_type(w0 & _HI, jnp.float32)
                for r in range(1, NNEI):
                    wr = rows[slot, p * NNEI + r, pl.ds(d * LANES, LANES)]
                    acc_lo = acc_lo + lax.bitcast_convert_type(
                        wr << 16, jnp.float32)
                    acc_hi = acc_hi + lax.bitcast_convert_type(
                        wr & _HI, jnp.float32)
                nbuf[slot, p, pl.ds(d * LANES, LANES)] = acc_lo
                nbuf[slot, p, pl.ds(W64 + d * LANES, LANES)] = acc_hi
            return carry
        lax.fori_loop(0, P, pos, 0)

    fire_nei(0, 0)
    fire_nei(1, 1)

    def process(i, c, slot):
        wait_nei(c, slot)

        @pl.when(i > 0)
        def _():
            pltpu.make_async_copy(nbuf.at[slot], nei_out.at[tbase],
                                  osem.at[slot]).wait()

        reduce_chunk(slot, c)
        pltpu.async_copy(nbuf.at[slot], nei_out.at[tbase + c],
                         osem.at[slot])

        @pl.when(i < NCH // 2 - 1)
        def _():
            fire_nei(c + 2, slot)

    def pair_body(i, carry):
        process(i, 2 * i, 0)
        process(i, 2 * i + 1, 1)
        return carry

    lax.fori_loop(0, NCH // 2, pair_body, 0)

    pltpu.make_async_copy(nbuf.at[0], nei_out.at[tbase], osem.at[0]).wait()
    pltpu.make_async_copy(nbuf.at[1], nei_out.at[tbase], osem.at[1]).wait()


_sc_nei = functools.partial(
    pl.kernel,
    out_type=jax.ShapeDtypeStruct((N // 8, 8, D), jnp.float32),
    mesh=plsc.VectorSubcoreMesh(core_axis_name="c", subcore_axis_name="s"),
    compiler_params=pltpu.CompilerParams(use_tc_tiling_on_sc=False),
    scratch_types=[
        pltpu.VMEM((PPW * NNEI,), jnp.int32),
        pltpu.VMEM((2, P * NNEI, W64), jnp.int32),
        pltpu.VMEM((2, P, D), jnp.float32),
        pltpu.SemaphoreType.DMA((2,)),
        pltpu.SemaphoreType.DMA((2,)),
    ],
)(_sc_nei_body)


def _mm_body(self_ref, nei_ref, w0a, w0s, w1a, w1s, out_ref):
    blk = out_ref.shape[0]
    aggr = nei_ref[...].reshape(blk, D) * (1.0 / NNEI)
    selfx = self_ref[...]
    h = jnp.maximum(
        jnp.dot(selfx, w0s[...], preferred_element_type=jnp.float32)
        + jnp.dot(aggr, w0a[...], preferred_element_type=jnp.float32), 0.0)
    out_ref[...] = jnp.maximum(
        jnp.dot(h, w1s[...], preferred_element_type=jnp.float32)
        + jnp.dot(aggr, w1a[...], preferred_element_type=jnp.float32), 0.0)


_MM_R = 4096
_mm = pl.pallas_call(
    _mm_body,
    out_shape=jax.ShapeDtypeStruct((N, D), jnp.float32),
    grid=(N // _MM_R,),
    in_specs=[
        pl.BlockSpec((_MM_R, D), lambda i: (i, 0)),
        pl.BlockSpec((_MM_R // 8, 8, D), lambda i: (i, 0, 0)),
        pl.BlockSpec((D, D), lambda i: (0, 0)),
        pl.BlockSpec((D, D), lambda i: (0, 0)),
        pl.BlockSpec((D, D), lambda i: (0, 0)),
        pl.BlockSpec((D, D), lambda i: (0, 0)),
    ],
    out_specs=pl.BlockSpec((_MM_R, D), lambda i: (i, 0)),
)


def kernel(adj_org, Emb, W0_agg, W0_self, W1_agg, W1_self):
    lo = lax.bitcast_convert_type(
        Emb[:, :W64].astype(jnp.bfloat16), jnp.uint16).astype(jnp.uint32)
    hi = lax.bitcast_convert_type(
        Emb[:, W64:].astype(jnp.bfloat16), jnp.uint16).astype(jnp.uint32)
    emb_i32 = lax.bitcast_convert_type(lo | (hi << 16), jnp.int32)

    adj = adj_org.reshape(N, K).astype(jnp.int32)
    idx_self = adj[:, 0].reshape(N // 128, 128)
    idx_nei = adj[:, 1:].reshape(N * NNEI)
    self_rows = _sc_self(Emb, idx_self)
    nei3 = _sc_nei(emb_i32, idx_nei)
    out = _mm(self_rows, nei3, W0_agg, W0_self, W1_agg, W1_self)
    return out.reshape(B, S1, S2, D)


# speedup vs baseline: 2.8699x; 1.1380x over previous
import functools

import jax
import jax.numpy as jnp
from jax import lax
from jax.experimental import pallas as pl
from jax.experimental.pallas import tpu as pltpu
from jax.experimental.pallas import tpu_sc as plsc

B, S1, S2, K = 1024, 1, 32, 17
D = 128
N = B * S1 * S2
NNEI = K - 1

NC, NS = 2, 16
NW = NC * NS
PPW = N // NW
P = 8
NCH = PPW // P
SCHUNK = 128
NSCH = PPW // SCHUNK
LANES = 16


def _sc_body(emb_h, idxn_h, idxs_h, self_out, nei_out,
             idxn_v, idxs_v, rows, nbuf, srows, gsem, osem, ssem, sosem):
    cid = lax.axis_index("c")
    sid = lax.axis_index("s")
    w = sid * NC + cid
    base = w * PPW

    pltpu.sync_copy(idxn_h.at[pl.ds(w * NCH, NCH)], idxn_v)
    pltpu.sync_copy(idxs_h.at[pl.ds(w * NSCH, NSCH)], idxs_v)

    def fire_nei(c, slot):
        pltpu.async_copy(emb_h.at[idxn_v.at[c]], rows.at[slot], gsem.at[slot])

    def wait_nei(c, slot):
        pltpu.make_async_copy(emb_h.at[idxn_v.at[c]], rows.at[slot],
                              gsem.at[slot]).wait()

    def fire_self(j):
        pltpu.async_copy(emb_h.at[idxs_v.at[j]], srows, ssem)

    def harvest_self(j):
        pltpu.make_async_copy(emb_h.at[idxs_v.at[j]], srows, ssem).wait()
        dst = self_out.at[pl.ds(base + j * SCHUNK, SCHUNK)]
        pltpu.async_copy(srows, dst, sosem)
        pltpu.make_async_copy(srows, dst, sosem).wait()

    def reduce_chunk(slot, c):
        def pos(p, carry):
            for d in range(D // LANES):
                acc = rows[slot, p * NNEI, pl.ds(d * LANES, LANES)]
                for r in range(1, NNEI):
                    acc = acc + rows[slot, p * NNEI + r, pl.ds(d * LANES, LANES)]
                nbuf[slot, p, pl.ds(d * LANES, LANES)] = acc
            return carry
        lax.fori_loop(0, P, pos, 0)

    fire_nei(0, 0)
    fire_nei(1, 1)
    fire_self(0)

    def process(i, c, slot):
        wait_nei(c, slot)

        @pl.when(i > 0)
        def _():
            pltpu.make_async_copy(nbuf.at[slot],
                                  nei_out.at[pl.ds(base, P)],
                                  osem.at[slot]).wait()

        reduce_chunk(slot, c)
        pltpu.async_copy(nbuf.at[slot],
                         nei_out.at[pl.ds(base + c * P, P)],
                         osem.at[slot])

        @pl.when(i < NCH // 2 - 1)
        def _():
            fire_nei(c + 2, slot)

    def pair_body(i, carry):
        @pl.when(i % 8 == 4)
        def _():
            j = i // 8
            harvest_self(j)

            @pl.when(j + 1 < NSCH)
            def _():
                fire_self(j + 1)

        process(i, 2 * i, 0)
        process(i, 2 * i + 1, 1)
        return carry

    lax.fori_loop(0, NCH // 2, pair_body, 0)

    pltpu.make_async_copy(nbuf.at[0], nei_out.at[pl.ds(base, P)],
                          osem.at[0]).wait()
    pltpu.make_async_copy(nbuf.at[1], nei_out.at[pl.ds(base, P)],
                          osem.at[1]).wait()


_sc_gather_reduce = functools.partial(
    pl.kernel,
    out_type=(jax.ShapeDtypeStruct((N, D), jnp.float32),
              jax.ShapeDtypeStruct((N, D), jnp.float32)),
    mesh=plsc.VectorSubcoreMesh(core_axis_name="c", subcore_axis_name="s"),
    scratch_types=[
        pltpu.VMEM((NCH, 128), jnp.int32),
        pltpu.VMEM((NSCH, 128), jnp.int32),
        pltpu.VMEM((2, P * NNEI, D), jnp.float32),
        pltpu.VMEM((2, P, D), jnp.float32),
        pltpu.VMEM((SCHUNK, D), jnp.float32),
        pltpu.SemaphoreType.DMA((2,)),
        pltpu.SemaphoreType.DMA((2,)),
        pltpu.SemaphoreType.DMA,
        pltpu.SemaphoreType.DMA,
    ],
)(_sc_body)


def _mm_body(self_ref, nei_ref, w0a, w0s, w1a, w1s, out_ref):
    aggr = nei_ref[...] * (1.0 / NNEI)
    h = jnp.maximum(
        jnp.dot(self_ref[...], w0s[...], preferred_element_type=jnp.float32)
        + jnp.dot(aggr, w0a[...], preferred_element_type=jnp.float32), 0.0)
    out_ref[...] = jnp.maximum(
        jnp.dot(h, w1s[...], preferred_element_type=jnp.float32)
        + jnp.dot(aggr, w1a[...], preferred_element_type=jnp.float32), 0.0)


_MM_R = 4096
_mm = pl.pallas_call(
    _mm_body,
    out_shape=jax.ShapeDtypeStruct((N, D), jnp.float32),
    grid=(N // _MM_R,),
    in_specs=[
        pl.BlockSpec((_MM_R, D), lambda i: (i, 0)),
        pl.BlockSpec((_MM_R, D), lambda i: (i, 0)),
        pl.BlockSpec((D, D), lambda i: (0, 0)),
        pl.BlockSpec((D, D), lambda i: (0, 0)),
        pl.BlockSpec((D, D), lambda i: (0, 0)),
        pl.BlockSpec((D, D), lambda i: (0, 0)),
    ],
    out_specs=pl.BlockSpec((_MM_R, D), lambda i: (i, 0)),
)


def kernel(adj_org, Emb, W0_agg, W0_self, W1_agg, W1_self):
    adj = adj_org.reshape(N, K).astype(jnp.int32)
    idx_self = adj[:, 0].reshape(N // 128, 128)
    idx_nei = adj[:, 1:].reshape(N * NNEI // 128, 128)
    self_rows, nei_sum = _sc_gather_reduce(Emb, idx_nei, idx_self)
    out = _mm(self_rows, nei_sum, W0_agg, W0_self, W1_agg, W1_self)
    return out.reshape(B, S1, S2, D)
